# baseline (speedup 1.0000x reference)
import functools

import jax
import jax.numpy as jnp
import numpy as _np
from jax import lax
from jax.experimental import pallas as pl
from jax.experimental.pallas import tpu as pltpu
from jax.experimental.pallas import tpu_sc as plsc

N = 10000
H = 32
CIN = 128
COUT = 128
K = 15
SIGMA = 2.0
GROUPS = 32
NEG_SLOPE = 0.1
EPS = 1e-5

NW = 32
PPW = 320
NPAD = NW * PPW
KGROUPS = ((0, 6), (6, 6), (12, 3))


def _sqrt16(x):
    xi = lax.bitcast_convert_type(x, jnp.int32)
    yi = jnp.int32(0x5F3759DF) - (xi >> 1)
    y = lax.bitcast_convert_type(yi, jnp.float32)
    for _ in range(2):
        y = y * (1.5 - 0.5 * x * y * y)
    return x * y


def _sc_body(feats, px, py, pz, rs, qp, idxp, kp,
             out_w, out_nn,
             px_v, py_v, pz_v, r_v, q_v, idx_v, kp_v, w_v, hl_v, nn_v,
             rows0, rows1, wacc0, wacc1, gsem0, gsem1, osem0, osem1):
    wid = lax.axis_index("s") * 2 + lax.axis_index("c")
    base = wid * PPW
    pltpu.sync_copy(px, px_v)
    pltpu.sync_copy(py, py_v)
    pltpu.sync_copy(pz, pz_v)
    pltpu.sync_copy(rs, r_v)
    pltpu.sync_copy(qp.at[pl.ds(base * 16, PPW * 16)], q_v)
    pltpu.sync_copy(idxp.at[pl.ds(base * H, PPW * H)], idx_v.at[pl.ds(0, PPW * H)])
    pltpu.sync_copy(kp, kp_v)
    zi = jnp.zeros((16,), jnp.int32)
    idx_v[pl.ds(PPW * H, 16)] = zi
    idx_v[pl.ds(PPW * H + 16, 16)] = zi

    rows = (rows0, rows1)
    waccs = (wacc0, wacc1)
    gsems = (gsem0, gsem1)
    osems = (osem0, osem1)

    pltpu.async_copy(feats.at[idx_v.at[pl.ds(0, H)]], rows0, gsem0)

    def do_point(i, b):
        rows_b = rows[b]
        wacc_b = waccs[b]
        pltpu.async_copy(feats.at[idx_v.at[pl.ds((i + 1) * H, H)]],
                         rows[1 - b], gsems[1 - b])

        iv0 = idx_v[pl.ds(i * H, 16)]
        iv1 = idx_v[pl.ds(i * H + 16, 16)]
        qrow = q_v[pl.ds(i * 16, 16)]
        qx = qrow[0]
        qy = qrow[1]
        qz = qrow[2]
        nx0 = plsc.load_gather(px_v, [iv0]) - qx
        ny0 = plsc.load_gather(py_v, [iv0]) - qy
        nz0 = plsc.load_gather(pz_v, [iv0]) - qz
        nx1 = plsc.load_gather(px_v, [iv1]) - qx
        ny1 = plsc.load_gather(py_v, [iv1]) - qy
        nz1 = plsc.load_gather(pz_v, [iv1]) - qz

        rs0 = plsc.load_gather(r_v, [iv0])
        rs1 = plsc.load_gather(r_v, [iv1])
        cnt = (plsc.all_reduce_population_count(rs0 > 0.0)
               + plsc.all_reduce_population_count(rs1 > 0.0))
        cntf = jnp.maximum(cnt.astype(jnp.float32), 1.0)
        nn_v[pl.ds(i * 16, 16)] = cntf

        kpxv = kp_v[pl.ds(0, 16)]
        kpyv = kp_v[pl.ds(16, 16)]
        kpzv = kp_v[pl.ds(32, 16)]
        lane = lax.broadcasted_iota(jnp.int32, (16,), 0)
        for k in range(K):
            kpx = kpxv[k]
            kpy = kpyv[k]
            kpz = kpzv[k]
            kcol = jnp.full((16,), k, jnp.int32)
            for g, (ax, ay, az) in enumerate(((nx0, ny0, nz0), (nx1, ny1, nz1))):
                dx = ax - kpx
                dy = ay - kpy
                dz = az - kpz
                sq = dx * dx + dy * dy + dz * dz
                dd = _sqrt16(sq)
                w = jnp.maximum(1.0 - dd * (1.0 / SIGMA), 0.0)
                plsc.store_scatter(w_v, [(lane + g * 16) * 16 + kcol], w)

        pltpu.make_async_copy(feats.at[idx_v.at[pl.ds(i * H, H)]],
                              rows_b, gsems[b]).wait()

        for k0, ksz in KGROUPS:
            def hbody(h, acc, _k0=k0, _ksz=ksz):
                wrow = w_v[pl.ds(h * 16, 16)]
                ws = tuple(jnp.full((16,), wrow[_k0 + t2], jnp.float32)
                           for t2 in range(_ksz))
                rvs = tuple(rows_b[h, pl.ds(j * 16, 16)] for j in range(8))
                return tuple(acc[t2 * 8 + j] + ws[t2] * rvs[j]
                             for t2 in range(_ksz) for j in range(8))

            acc0 = tuple(jnp.zeros((16,), jnp.float32) for _ in range(ksz * 8))
            acc = lax.fori_loop(0, H, hbody, acc0)
            for t in range(ksz):
                for j in range(8):
                    wacc_b[pl.ds((k0 + t) * CIN + j * 16, 16)] = \
                        acc[t * 8 + j]

        @pl.when(i >= 2)
        def _():
            pltpu.make_async_copy(
                wacc_b, out_w.at[pl.ds((base + i - 2) * K * CIN, K * CIN)],
                osems[b]).wait()

        pltpu.async_copy(
            wacc_b, out_w.at[pl.ds((base + i) * K * CIN, K * CIN)], osems[b])

    def loop_body(i2, carry):
        do_point(i2 * 2, 0)
        do_point(i2 * 2 + 1, 1)
        return carry

    lax.fori_loop(0, PPW // 2, loop_body, 0)

    pltpu.make_async_copy(
        wacc0, out_w.at[pl.ds((base + PPW - 2) * K * CIN, K * CIN)],
        osem0).wait()
    pltpu.make_async_copy(
        wacc1, out_w.at[pl.ds((base + PPW - 1) * K * CIN, K * CIN)],
        osem1).wait()
    pltpu.make_async_copy(feats.at[idx_v.at[pl.ds(PPW * H, H)]],
                          rows0, gsem0).wait()
    pltpu.sync_copy(nn_v, out_nn.at[pl.ds(base * 16, PPW * 16)])


def _sc_stage(feats, px, py, pz, rsum, qpad, idxpad, kp_soa):
    mesh = plsc.VectorSubcoreMesh(core_axis_name="c", subcore_axis_name="s")
    f = pl.kernel(
        _sc_body,
        mesh=mesh,
        compiler_params=pltpu.CompilerParams(needs_layout_passes=False),
        out_type=[
            jax.ShapeDtypeStruct((NPAD * K * CIN,), jnp.float32),
            jax.ShapeDtypeStruct((NPAD * 16,), jnp.float32),
        ],
        scratch_types=[
            pltpu.VMEM((N,), jnp.float32),
            pltpu.VMEM((N,), jnp.float32),
            pltpu.VMEM((N,), jnp.float32),
            pltpu.VMEM((N,), jnp.float32),
            pltpu.VMEM((PPW * 16,), jnp.float32),
            pltpu.VMEM(((PPW + 1) * H,), jnp.int32),
            pltpu.VMEM((48,), jnp.float32),
            pltpu.VMEM((H * 16,), jnp.float32),
            pltpu.VMEM((48,), jnp.int32),
            pltpu.VMEM((PPW * 16,), jnp.float32),
            pltpu.VMEM((H, CIN), jnp.float32),
            pltpu.VMEM((H, CIN), jnp.float32),
            pltpu.VMEM((K * CIN,), jnp.float32),
            pltpu.VMEM((K * CIN,), jnp.float32),
            pltpu.SemaphoreType.DMA,
            pltpu.SemaphoreType.DMA,
            pltpu.SemaphoreType.DMA,
            pltpu.SemaphoreType.DMA,
        ],
    )
    return f(feats, px, py, pz, rsum, qpad, idxpad, kp_soa)


def _rowsum_stage(feats):
    def body(f_ref, r_ref):
        r_ref[...] = jnp.sum(f_ref[...], axis=1, keepdims=True)

    return pl.pallas_call(
        body,
        out_shape=jax.ShapeDtypeStruct((N, 1), jnp.float32),
    )(feats)


def _mm_stats_stage(w2d, wf, nn, bias2d):
    B = 1024
    nblk = NPAD // B

    def body(w_ref, wf_ref, nn_ref, b_ref, x_ref, s_ref, q_ref):
        i = pl.program_id(0)
        mm = jnp.dot(w_ref[...], wf_ref[...],
                     preferred_element_type=jnp.float32)
        x = mm / nn_ref[...] + b_ref[...]
        x_ref[...] = x
        gid = i * B + lax.broadcasted_iota(jnp.int32, (B, 1), 0)
        xm = jnp.where(gid < N, x, 0.0)

        @pl.when(i == 0)
        def _():
            s_ref[...] = jnp.zeros_like(s_ref)
            q_ref[...] = jnp.zeros_like(q_ref)

        s_ref[...] += jnp.sum(xm, axis=0, keepdims=True)
        q_ref[...] += jnp.sum(xm * xm, axis=0, keepdims=True)

    return pl.pallas_call(
        body,
        grid=(nblk,),
        in_specs=[
            pl.BlockSpec((B, K * CIN), lambda i: (i, 0)),
            pl.BlockSpec((K * CIN, COUT), lambda i: (0, 0)),
            pl.BlockSpec((B, 1), lambda i: (i, 0)),
            pl.BlockSpec((1, COUT), lambda i: (0, 0)),
        ],
        out_specs=[
            pl.BlockSpec((B, COUT), lambda i: (i, 0)),
            pl.BlockSpec((1, COUT), lambda i: (0, 0)),
            pl.BlockSpec((1, COUT), lambda i: (0, 0)),
        ],
        out_shape=[
            jax.ShapeDtypeStruct((NPAD, COUT), jnp.float32),
            jax.ShapeDtypeStruct((1, COUT), jnp.float32),
            jax.ShapeDtypeStruct((1, COUT), jnp.float32),
        ],
    )(w2d, wf, nn, bias2d)


def _norm_stage(x, s, q, gamma2d, beta2d):
    B = 1000

    def body(x_ref, s_ref, q_ref, g_ref, b_ref, y_ref):
        gsz = COUT // GROUPS
        ii = lax.broadcasted_iota(jnp.int32, (COUT, COUT), 0) // gsz
        jj = lax.broadcasted_iota(jnp.int32, (COUT, COUT), 1) // gsz
        gmat = (ii == jj).astype(jnp.float32)
        denom = float(gsz * N)
        mean = jnp.dot(s_ref[...], gmat,
                       preferred_element_type=jnp.float32) / denom
        e2 = jnp.dot(q_ref[...], gmat,
                     preferred_element_type=jnp.float32) / denom
        var = e2 - mean * mean
        scale = g_ref[...] * lax.rsqrt(var + EPS)
        shift = b_ref[...] - mean * scale
        y = x_ref[...] * scale + shift
        y_ref[...] = jnp.where(y >= 0.0, y, NEG_SLOPE * y)

    return pl.pallas_call(
        body,
        grid=(N // B,),
        in_specs=[
            pl.BlockSpec((B, COUT), lambda i: (i, 0)),
            pl.BlockSpec((1, COUT), lambda i: (0, 0)),
            pl.BlockSpec((1, COUT), lambda i: (0, 0)),
            pl.BlockSpec((1, COUT), lambda i: (0, 0)),
            pl.BlockSpec((1, COUT), lambda i: (0, 0)),
        ],
        out_specs=pl.BlockSpec((B, COUT), lambda i: (i, 0)),
        out_shape=jax.ShapeDtypeStruct((N, COUT), jnp.float32),
    )(x, s, q, gamma2d, beta2d)


def kernel(s_feats, q_points, s_points, neighbor_indices, weights, bias,
           gamma, beta, kernel_points):
    f32 = jnp.float32
    rsum = _rowsum_stage(s_feats).reshape(N)

    px = s_points[:, 0].astype(f32)
    py = s_points[:, 1].astype(f32)
    pz = s_points[:, 2].astype(f32)
    qpad = jnp.pad(q_points.astype(f32), ((0, NPAD - N), (0, 13))).reshape(-1)
    idxpad = jnp.concatenate(
        [neighbor_indices.astype(jnp.int32),
         jnp.zeros((NPAD - N, H), jnp.int32)], axis=0).reshape(-1)
    kp_soa = jnp.concatenate(
        [kernel_points.astype(f32).T, jnp.zeros((3, 16 - K), f32)],
        axis=1).reshape(-1)

    weighted, nnflat = _sc_stage(s_feats.astype(f32), px, py, pz, rsum,
                                 qpad, idxpad, kp_soa)

    w2d = weighted.reshape(NPAD, K * CIN)
    wf = weights.astype(f32).reshape(K * CIN, COUT)
    nn = nnflat.reshape(NPAD, 16)[:, :1]
    x, s, q = _mm_stats_stage(w2d, wf, nn, bias.astype(f32).reshape(1, COUT))
    y = _norm_stage(x, s, q, gamma.astype(f32).reshape(1, COUT),
                    beta.astype(f32).reshape(1, COUT))
    return y[:, None, :]


# speedup vs baseline: 1.3363x; 1.3363x over previous
import functools

import jax
import jax.numpy as jnp
import numpy as _np
from jax import lax
from jax.experimental import pallas as pl
from jax.experimental.pallas import tpu as pltpu
from jax.experimental.pallas import tpu_sc as plsc

N = 10000
H = 32
CIN = 128
COUT = 128
K = 15
SIGMA = 2.0
GROUPS = 32
NEG_SLOPE = 0.1
EPS = 1e-5

NW = 32
PPW = 320
NPAD = NW * PPW
KGROUPS = ((0, 4), (4, 4), (8, 4), (12, 3))


def _sqrt16(x):
    xi = lax.bitcast_convert_type(x, jnp.int32)
    yi = jnp.int32(0x5F3759DF) - (xi >> 1)
    y = lax.bitcast_convert_type(yi, jnp.float32)
    for _ in range(2):
        y = y * (1.5 - 0.5 * x * y * y)
    return x * y


def _sc_body(feats, px, py, pz, rs, qp, idxp, kp,
             out_w, out_nn,
             px_v, py_v, pz_v, r_v, q_v, idx_v, kp_v, w_v, hl_v, nn_v,
             rows0, rows1, wacc0, wacc1, gsem0, gsem1, osem0, osem1):
    wid = lax.axis_index("s") * 2 + lax.axis_index("c")
    base = wid * PPW
    pltpu.sync_copy(px, px_v)
    pltpu.sync_copy(py, py_v)
    pltpu.sync_copy(pz, pz_v)
    pltpu.sync_copy(rs, r_v)
    pltpu.sync_copy(qp.at[pl.ds(base * 16, PPW * 16)], q_v)
    pltpu.sync_copy(idxp.at[pl.ds(base * H, PPW * H)], idx_v.at[pl.ds(0, PPW * H)])
    pltpu.sync_copy(kp, kp_v)
    zi = jnp.zeros((16,), jnp.int32)
    idx_v[pl.ds(PPW * H, 16)] = zi
    idx_v[pl.ds(PPW * H + 16, 16)] = zi

    rows = (rows0, rows1)
    waccs = (wacc0, wacc1)
    gsems = (gsem0, gsem1)
    osems = (osem0, osem1)

    pltpu.async_copy(feats.at[idx_v.at[pl.ds(0, H)]], rows0, gsem0)

    def do_point(i, b):
        rows_b = rows[b]
        wacc_b = waccs[b]
        pltpu.async_copy(feats.at[idx_v.at[pl.ds((i + 1) * H, H)]],
                         rows[1 - b], gsems[1 - b])

        iv0 = idx_v[pl.ds(i * H, 16)]
        iv1 = idx_v[pl.ds(i * H + 16, 16)]
        qrow = q_v[pl.ds(i * 16, 16)]
        qx = qrow[0]
        qy = qrow[1]
        qz = qrow[2]
        nx0 = plsc.load_gather(px_v, [iv0]) - qx
        ny0 = plsc.load_gather(py_v, [iv0]) - qy
        nz0 = plsc.load_gather(pz_v, [iv0]) - qz
        nx1 = plsc.load_gather(px_v, [iv1]) - qx
        ny1 = plsc.load_gather(py_v, [iv1]) - qy
        nz1 = plsc.load_gather(pz_v, [iv1]) - qz

        rs0 = plsc.load_gather(r_v, [iv0])
        rs1 = plsc.load_gather(r_v, [iv1])
        cnt = (plsc.all_reduce_population_count(rs0 > 0.0)
               + plsc.all_reduce_population_count(rs1 > 0.0))
        cntf = jnp.maximum(cnt.astype(jnp.float32), 1.0)
        nn_v[pl.ds(i * 16, 16)] = cntf

        kpxv = kp_v[pl.ds(0, 16)]
        kpyv = kp_v[pl.ds(16, 16)]
        kpzv = kp_v[pl.ds(32, 16)]
        lane = lax.broadcasted_iota(jnp.int32, (16,), 0)
        for k in range(K):
            kpx = kpxv[k]
            kpy = kpyv[k]
            kpz = kpzv[k]
            kcol = jnp.full((16,), k, jnp.int32)
            for g, (ax, ay, az) in enumerate(((nx0, ny0, nz0), (nx1, ny1, nz1))):
                dx = ax - kpx
                dy = ay - kpy
                dz = az - kpz
                sq = dx * dx + dy * dy + dz * dz
                dd = _sqrt16(sq)
                w = jnp.maximum(1.0 - dd * (1.0 / SIGMA), 0.0)
                plsc.store_scatter(w_v, [(lane + g * 16) * 16 + kcol], w)

        pltpu.make_async_copy(feats.at[idx_v.at[pl.ds(i * H, H)]],
                              rows_b, gsems[b]).wait()

        for k0, ksz in KGROUPS:
            def hbody(h, acc, _k0=k0, _ksz=ksz):
                wrow = w_v[pl.ds(h * 16, 16)]
                ws = tuple(jnp.full((16,), wrow[_k0 + t2], jnp.float32)
                           for t2 in range(_ksz))
                rvs = tuple(rows_b[h, pl.ds(j * 16, 16)] for j in range(8))
                return tuple(acc[t2 * 8 + j] + ws[t2] * rvs[j]
                             for t2 in range(_ksz) for j in range(8))

            acc0 = tuple(jnp.zeros((16,), jnp.float32) for _ in range(ksz * 8))
            acc = lax.fori_loop(0, H, hbody, acc0)
            for t in range(ksz):
                for j in range(8):
                    wacc_b[pl.ds((k0 + t) * CIN + j * 16, 16)] = \
                        acc[t * 8 + j]

        @pl.when(i >= 2)
        def _():
            pltpu.make_async_copy(
                wacc_b, out_w.at[pl.ds((base + i - 2) * K * CIN, K * CIN)],
                osems[b]).wait()

        pltpu.async_copy(
            wacc_b, out_w.at[pl.ds((base + i) * K * CIN, K * CIN)], osems[b])

    def loop_body(i2, carry):
        do_point(i2 * 2, 0)
        do_point(i2 * 2 + 1, 1)
        return carry

    lax.fori_loop(0, PPW // 2, loop_body, 0)

    pltpu.make_async_copy(
        wacc0, out_w.at[pl.ds((base + PPW - 2) * K * CIN, K * CIN)],
        osem0).wait()
    pltpu.make_async_copy(
        wacc1, out_w.at[pl.ds((base + PPW - 1) * K * CIN, K * CIN)],
        osem1).wait()
    pltpu.make_async_copy(feats.at[idx_v.at[pl.ds(PPW * H, H)]],
                          rows0, gsem0).wait()
    pltpu.sync_copy(nn_v, out_nn.at[pl.ds(base * 16, PPW * 16)])


def _sc_stage(feats, px, py, pz, rsum, qpad, idxpad, kp_soa):
    mesh = plsc.VectorSubcoreMesh(core_axis_name="c", subcore_axis_name="s")
    f = pl.kernel(
        _sc_body,
        mesh=mesh,
        compiler_params=pltpu.CompilerParams(needs_layout_passes=False),
        out_type=[
            jax.ShapeDtypeStruct((NPAD * K * CIN,), jnp.float32),
            jax.ShapeDtypeStruct((NPAD * 16,), jnp.float32),
        ],
        scratch_types=[
            pltpu.VMEM((N,), jnp.float32),
            pltpu.VMEM((N,), jnp.float32),
            pltpu.VMEM((N,), jnp.float32),
            pltpu.VMEM((N,), jnp.float32),
            pltpu.VMEM((PPW * 16,), jnp.float32),
            pltpu.VMEM(((PPW + 1) * H,), jnp.int32),
            pltpu.VMEM((48,), jnp.float32),
            pltpu.VMEM((H * 16,), jnp.float32),
            pltpu.VMEM((48,), jnp.int32),
            pltpu.VMEM((PPW * 16,), jnp.float32),
            pltpu.VMEM((H, CIN), jnp.float32),
            pltpu.VMEM((H, CIN), jnp.float32),
            pltpu.VMEM((K * CIN,), jnp.float32),
            pltpu.VMEM((K * CIN,), jnp.float32),
            pltpu.SemaphoreType.DMA,
            pltpu.SemaphoreType.DMA,
            pltpu.SemaphoreType.DMA,
            pltpu.SemaphoreType.DMA,
        ],
    )
    return f(feats, px, py, pz, rsum, qpad, idxpad, kp_soa)


def _rowsum_stage(feats):
    def body(f_ref, r_ref):
        r_ref[...] = jnp.sum(f_ref[...], axis=1, keepdims=True)

    return pl.pallas_call(
        body,
        out_shape=jax.ShapeDtypeStruct((N, 1), jnp.float32),
    )(feats)


def _mm_stats_stage(w2d, wf, nn, bias2d):
    B = 2048
    nblk = NPAD // B

    def body(w_ref, wf_ref, nn_ref, b_ref, x_ref, s_ref, q_ref):
        i = pl.program_id(0)
        mm = jnp.dot(w_ref[...], wf_ref[...],
                     preferred_element_type=jnp.float32)
        x = mm / nn_ref[...] + b_ref[...]
        x_ref[...] = x
        gid = i * B + lax.broadcasted_iota(jnp.int32, (B, 1), 0)
        xm = jnp.where(gid < N, x, 0.0)

        @pl.when(i == 0)
        def _():
            s_ref[...] = jnp.zeros_like(s_ref)
            q_ref[...] = jnp.zeros_like(q_ref)

        s_ref[...] += jnp.sum(xm, axis=0, keepdims=True)
        q_ref[...] += jnp.sum(xm * xm, axis=0, keepdims=True)

    return pl.pallas_call(
        body,
        grid=(nblk,),
        in_specs=[
            pl.BlockSpec((B, K * CIN), lambda i: (i, 0)),
            pl.BlockSpec((K * CIN, COUT), lambda i: (0, 0)),
            pl.BlockSpec((B, 1), lambda i: (i, 0)),
            pl.BlockSpec((1, COUT), lambda i: (0, 0)),
        ],
        out_specs=[
            pl.BlockSpec((B, COUT), lambda i: (i, 0)),
            pl.BlockSpec((1, COUT), lambda i: (0, 0)),
            pl.BlockSpec((1, COUT), lambda i: (0, 0)),
        ],
        out_shape=[
            jax.ShapeDtypeStruct((NPAD, COUT), jnp.float32),
            jax.ShapeDtypeStruct((1, COUT), jnp.float32),
            jax.ShapeDtypeStruct((1, COUT), jnp.float32),
        ],
    )(w2d, wf, nn, bias2d)


def _norm_stage(x, s, q, gamma2d, beta2d):
    B = 2000

    def body(x_ref, s_ref, q_ref, g_ref, b_ref, y_ref):
        gsz = COUT // GROUPS
        ii = lax.broadcasted_iota(jnp.int32, (COUT, COUT), 0) // gsz
        jj = lax.broadcasted_iota(jnp.int32, (COUT, COUT), 1) // gsz
        gmat = (ii == jj).astype(jnp.float32)
        denom = float(gsz * N)
        mean = jnp.dot(s_ref[...], gmat,
                       preferred_element_type=jnp.float32) / denom
        e2 = jnp.dot(q_ref[...], gmat,
                     preferred_element_type=jnp.float32) / denom
        var = e2 - mean * mean
        scale = g_ref[...] * lax.rsqrt(var + EPS)
        shift = b_ref[...] - mean * scale
        y = x_ref[...] * scale + shift
        y_ref[...] = jnp.where(y >= 0.0, y, NEG_SLOPE * y)

    return pl.pallas_call(
        body,
        grid=(N // B,),
        in_specs=[
            pl.BlockSpec((B, COUT), lambda i: (i, 0)),
            pl.BlockSpec((1, COUT), lambda i: (0, 0)),
            pl.BlockSpec((1, COUT), lambda i: (0, 0)),
            pl.BlockSpec((1, COUT), lambda i: (0, 0)),
            pl.BlockSpec((1, COUT), lambda i: (0, 0)),
        ],
        out_specs=pl.BlockSpec((B, COUT), lambda i: (i, 0)),
        out_shape=jax.ShapeDtypeStruct((N, COUT), jnp.float32),
    )(x, s, q, gamma2d, beta2d)


def kernel(s_feats, q_points, s_points, neighbor_indices, weights, bias,
           gamma, beta, kernel_points):
    f32 = jnp.float32
    rsum = _rowsum_stage(s_feats).reshape(N)

    px = s_points[:, 0].astype(f32)
    py = s_points[:, 1].astype(f32)
    pz = s_points[:, 2].astype(f32)
    qpad = jnp.pad(q_points.astype(f32), ((0, NPAD - N), (0, 13))).reshape(-1)
    idxpad = jnp.concatenate(
        [neighbor_indices.astype(jnp.int32),
         jnp.zeros((NPAD - N, H), jnp.int32)], axis=0).reshape(-1)
    kp_soa = jnp.concatenate(
        [kernel_points.astype(f32).T, jnp.zeros((3, 16 - K), f32)],
        axis=1).reshape(-1)

    weighted, nnflat = _sc_stage(s_feats.astype(f32), px, py, pz, rsum,
                                 qpad, idxpad, kp_soa)

    w2d = weighted.reshape(NPAD, K * CIN)
    wf = weights.astype(f32).reshape(K * CIN, COUT)
    nn = nnflat.reshape(NPAD, 16)[:, :1]
    x, s, q = _mm_stats_stage(w2d, wf, nn, bias.astype(f32).reshape(1, COUT))
    y = _norm_stage(x, s, q, gamma.astype(f32).reshape(1, COUT),
                    beta.astype(f32).reshape(1, COUT))
    return y[:, None, :]


# speedup vs baseline: 1.3378x; 1.0012x over previous
import jax
import jax.numpy as jnp
import numpy as _np
from jax import lax
from jax.experimental import pallas as pl
from jax.experimental.pallas import tpu as pltpu
from jax.experimental.pallas import tpu_sc as plsc

N = 10000
H = 32
CIN = 128
COUT = 128
K = 15
SIGMA = 2.0
GROUPS = 32
NEG_SLOPE = 0.1
EPS = 1e-5

NW = 32
PPW = 320
NPAD = NW * PPW
KGROUPS = ((0, 4), (4, 4), (8, 4), (12, 3))


def _sqrt16(x):
    xi = lax.bitcast_convert_type(x, jnp.int32)
    yi = jnp.int32(0x5F3759DF) - (xi >> 1)
    y = lax.bitcast_convert_type(yi, jnp.float32)
    for _ in range(2):
        y = y * (1.5 - 0.5 * x * y * y)
    return x * y


def _sc_body(feats, px, py, pz, rs, qp, idxp, kp,
             out_w, out_nn,
             px_v, py_v, pz_v, r_v, q_v, idx_v, kp_v, w_v, nn_v,
             rows0, rows1, wacc0, wacc1, gsem0, gsem1, osem0, osem1):
    wid = lax.axis_index("s") * 2 + lax.axis_index("c")
    base = wid * PPW
    pltpu.sync_copy(px, px_v)
    pltpu.sync_copy(py, py_v)
    pltpu.sync_copy(pz, pz_v)
    pltpu.sync_copy(rs, r_v)
    pltpu.sync_copy(qp.at[pl.ds(base * 16, PPW * 16)], q_v)
    pltpu.sync_copy(idxp.at[pl.ds(base * H, PPW * H)], idx_v.at[pl.ds(0, PPW * H)])
    pltpu.sync_copy(kp, kp_v)
    zi = jnp.zeros((16,), jnp.int32)
    idx_v[pl.ds(PPW * H, 16)] = zi
    idx_v[pl.ds(PPW * H + 16, 16)] = zi

    rows = (rows0, rows1)
    waccs = (wacc0, wacc1)
    gsems = (gsem0, gsem1)
    osems = (osem0, osem1)

    pltpu.async_copy(feats.at[idx_v.at[pl.ds(0, H)]], rows0, gsem0)

    def do_point(i, b):
        rows_b = rows[b]
        wacc_b = waccs[b]
        pltpu.async_copy(feats.at[idx_v.at[pl.ds((i + 1) * H, H)]],
                         rows[1 - b], gsems[1 - b])

        iv0 = idx_v[pl.ds(i * H, 16)]
        iv1 = idx_v[pl.ds(i * H + 16, 16)]
        qrow = q_v[pl.ds(i * 16, 16)]
        qx = qrow[0]
        qy = qrow[1]
        qz = qrow[2]
        nx0 = plsc.load_gather(px_v, [iv0]) - qx
        ny0 = plsc.load_gather(py_v, [iv0]) - qy
        nz0 = plsc.load_gather(pz_v, [iv0]) - qz
        nx1 = plsc.load_gather(px_v, [iv1]) - qx
        ny1 = plsc.load_gather(py_v, [iv1]) - qy
        nz1 = plsc.load_gather(pz_v, [iv1]) - qz

        rs0 = plsc.load_gather(r_v, [iv0])
        rs1 = plsc.load_gather(r_v, [iv1])
        cnt = (plsc.all_reduce_population_count(rs0 > 0.0)
               + plsc.all_reduce_population_count(rs1 > 0.0))
        cntf = jnp.maximum(cnt.astype(jnp.float32), 1.0)
        nn_v[pl.ds(i * 16, 16)] = cntf

        kpxv = kp_v[pl.ds(0, 16)]
        kpyv = kp_v[pl.ds(16, 16)]
        kpzv = kp_v[pl.ds(32, 16)]
        lane = lax.broadcasted_iota(jnp.int32, (16,), 0)
        for k in range(K):
            kpx = kpxv[k]
            kpy = kpyv[k]
            kpz = kpzv[k]
            kcol = jnp.full((16,), k, jnp.int32)
            for g, (ax, ay, az) in enumerate(((nx0, ny0, nz0), (nx1, ny1, nz1))):
                dx = ax - kpx
                dy = ay - kpy
                dz = az - kpz
                sq = dx * dx + dy * dy + dz * dz
                dd = _sqrt16(sq)
                w = jnp.maximum(1.0 - dd * (1.0 / SIGMA), 0.0)
                plsc.store_scatter(w_v, [(lane + g * 16) * 16 + kcol], w)

        pltpu.make_async_copy(feats.at[idx_v.at[pl.ds(i * H, H)]],
                              rows_b, gsems[b]).wait()

        for k0, ksz in KGROUPS:
            def hbody(h, acc, _k0=k0, _ksz=ksz):
                wrow = w_v[pl.ds(h * 16, 16)]
                ws = tuple(jnp.full((16,), wrow[_k0 + t2], jnp.float32)
                           for t2 in range(_ksz))
                rvs = tuple(rows_b[h, pl.ds(j * 16, 16)] for j in range(8))
                return tuple(acc[t2 * 8 + j] + ws[t2] * rvs[j]
                             for t2 in range(_ksz) for j in range(8))

            acc0 = tuple(jnp.zeros((16,), jnp.float32) for _ in range(ksz * 8))
            acc = lax.fori_loop(0, H, hbody, acc0)
            for t in range(ksz):
                for j in range(8):
                    wacc_b[pl.ds((k0 + t) * CIN + j * 16, 16)] = \
                        acc[t * 8 + j]

        @pl.when(i >= 2)
        def _():
            pltpu.make_async_copy(
                wacc_b, out_w.at[pl.ds((base + i - 2) * K * CIN, K * CIN)],
                osems[b]).wait()

        pltpu.async_copy(
            wacc_b, out_w.at[pl.ds((base + i) * K * CIN, K * CIN)], osems[b])

    def loop_body(i2, carry):
        do_point(i2 * 2, 0)
        do_point(i2 * 2 + 1, 1)
        return carry

    lax.fori_loop(0, PPW // 2, loop_body, 0)

    pltpu.make_async_copy(
        wacc0, out_w.at[pl.ds((base + PPW - 2) * K * CIN, K * CIN)],
        osem0).wait()
    pltpu.make_async_copy(
        wacc1, out_w.at[pl.ds((base + PPW - 1) * K * CIN, K * CIN)],
        osem1).wait()
    pltpu.make_async_copy(feats.at[idx_v.at[pl.ds(PPW * H, H)]],
                          rows0, gsem0).wait()
    pltpu.sync_copy(nn_v, out_nn.at[pl.ds(base * 16, PPW * 16)])


def _sc_stage(feats, px, py, pz, rsum, qpad, idxpad, kp_soa):
    mesh = plsc.VectorSubcoreMesh(core_axis_name="c", subcore_axis_name="s")
    f = pl.kernel(
        _sc_body,
        mesh=mesh,
        compiler_params=pltpu.CompilerParams(needs_layout_passes=False),
        out_type=[
            jax.ShapeDtypeStruct((NPAD * K * CIN,), jnp.float32),
            jax.ShapeDtypeStruct((NPAD * 16,), jnp.float32),
        ],
        scratch_types=[
            pltpu.VMEM((N,), jnp.float32),
            pltpu.VMEM((N,), jnp.float32),
            pltpu.VMEM((N,), jnp.float32),
            pltpu.VMEM((N,), jnp.float32),
            pltpu.VMEM((PPW * 16,), jnp.float32),
            pltpu.VMEM(((PPW + 1) * H,), jnp.int32),
            pltpu.VMEM((48,), jnp.float32),
            pltpu.VMEM((H * 16,), jnp.float32),
            pltpu.VMEM((PPW * 16,), jnp.float32),
            pltpu.VMEM((H, CIN), jnp.float32),
            pltpu.VMEM((H, CIN), jnp.float32),
            pltpu.VMEM((K * CIN,), jnp.float32),
            pltpu.VMEM((K * CIN,), jnp.float32),
            pltpu.SemaphoreType.DMA,
            pltpu.SemaphoreType.DMA,
            pltpu.SemaphoreType.DMA,
            pltpu.SemaphoreType.DMA,
        ],
    )
    return f(feats, px, py, pz, rsum, qpad, idxpad, kp_soa)


def _rowsum_stage(feats):
    def body(f_ref, r_ref):
        r_ref[...] = jnp.sum(f_ref[...], axis=1, keepdims=True)

    return pl.pallas_call(
        body,
        out_shape=jax.ShapeDtypeStruct((N, 1), jnp.float32),
    )(feats)


def _mm_stats_stage(w2d, wf, nn, bias2d):
    B = 2048
    nblk = NPAD // B

    def body(w_ref, wf_ref, nn_ref, b_ref, x_ref, s_ref, q_ref):
        i = pl.program_id(0)
        mm = jnp.dot(w_ref[...], wf_ref[...],
                     preferred_element_type=jnp.float32)
        x = mm / nn_ref[...] + b_ref[...]
        x_ref[...] = x
        gid = i * B + lax.broadcasted_iota(jnp.int32, (B, 1), 0)
        xm = jnp.where(gid < N, x, 0.0)

        @pl.when(i == 0)
        def _():
            s_ref[...] = jnp.zeros_like(s_ref)
            q_ref[...] = jnp.zeros_like(q_ref)

        s_ref[...] += jnp.sum(xm, axis=0, keepdims=True)
        q_ref[...] += jnp.sum(xm * xm, axis=0, keepdims=True)

    return pl.pallas_call(
        body,
        grid=(nblk,),
        in_specs=[
            pl.BlockSpec((B, K * CIN), lambda i: (i, 0)),
            pl.BlockSpec((K * CIN, COUT), lambda i: (0, 0)),
            pl.BlockSpec((B, 1), lambda i: (i, 0)),
            pl.BlockSpec((1, COUT), lambda i: (0, 0)),
        ],
        out_specs=[
            pl.BlockSpec((B, COUT), lambda i: (i, 0)),
            pl.BlockSpec((1, COUT), lambda i: (0, 0)),
            pl.BlockSpec((1, COUT), lambda i: (0, 0)),
        ],
        out_shape=[
            jax.ShapeDtypeStruct((NPAD, COUT), jnp.float32),
            jax.ShapeDtypeStruct((1, COUT), jnp.float32),
            jax.ShapeDtypeStruct((1, COUT), jnp.float32),
        ],
    )(w2d, wf, nn, bias2d)


def _norm_stage(x, s, q, gamma2d, beta2d):
    B = 2000

    def body(x_ref, s_ref, q_ref, g_ref, b_ref, y_ref):
        gsz = COUT // GROUPS
        ii = lax.broadcasted_iota(jnp.int32, (COUT, COUT), 0) // gsz
        jj = lax.broadcasted_iota(jnp.int32, (COUT, COUT), 1) // gsz
        gmat = (ii == jj).astype(jnp.float32)
        denom = float(gsz * N)
        mean = jnp.dot(s_ref[...], gmat,
                       preferred_element_type=jnp.float32) / denom
        e2 = jnp.dot(q_ref[...], gmat,
                     preferred_element_type=jnp.float32) / denom
        var = e2 - mean * mean
        scale = g_ref[...] * lax.rsqrt(var + EPS)
        shift = b_ref[...] - mean * scale
        y = x_ref[...] * scale + shift
        y_ref[...] = jnp.where(y >= 0.0, y, NEG_SLOPE * y)

    return pl.pallas_call(
        body,
        grid=(N // B,),
        in_specs=[
            pl.BlockSpec((B, COUT), lambda i: (i, 0)),
            pl.BlockSpec((1, COUT), lambda i: (0, 0)),
            pl.BlockSpec((1, COUT), lambda i: (0, 0)),
            pl.BlockSpec((1, COUT), lambda i: (0, 0)),
            pl.BlockSpec((1, COUT), lambda i: (0, 0)),
        ],
        out_specs=pl.BlockSpec((B, COUT), lambda i: (i, 0)),
        out_shape=jax.ShapeDtypeStruct((N, COUT), jnp.float32),
    )(x, s, q, gamma2d, beta2d)


def kernel(s_feats, q_points, s_points, neighbor_indices, weights, bias,
           gamma, beta, kernel_points):
    f32 = jnp.float32
    rsum = _rowsum_stage(s_feats).reshape(N)

    px = s_points[:, 0].astype(f32)
    py = s_points[:, 1].astype(f32)
    pz = s_points[:, 2].astype(f32)
    qpad = jnp.pad(q_points.astype(f32), ((0, NPAD - N), (0, 13))).reshape(-1)
    idxpad = jnp.concatenate(
        [neighbor_indices.astype(jnp.int32),
         jnp.zeros((NPAD - N, H), jnp.int32)], axis=0).reshape(-1)
    kp_soa = jnp.concatenate(
        [kernel_points.astype(f32).T, jnp.zeros((3, 16 - K), f32)],
        axis=1).reshape(-1)

    weighted, nnflat = _sc_stage(s_feats.astype(f32), px, py, pz, rsum,
                                 qpad, idxpad, kp_soa)

    w2d = weighted.reshape(NPAD, K * CIN)
    wf = weights.astype(f32).reshape(K * CIN, COUT)
    nn = nnflat.reshape(NPAD, 16)[:, :1]
    x, s, q = _mm_stats_stage(w2d, wf, nn, bias.astype(f32).reshape(1, COUT))
    y = _norm_stage(x, s, q, gamma.astype(f32).reshape(1, COUT),
                    beta.astype(f32).reshape(1, COUT))
    return y[:, None, :]


# speedup vs baseline: 1.3397x; 1.0014x over previous
import jax
import jax.numpy as jnp
from jax import lax
from jax.experimental import pallas as pl
from jax.experimental.pallas import tpu as pltpu
from jax.experimental.pallas import tpu_sc as plsc

N = 10000
H = 32
CIN = 128
COUT = 128
K = 15
SIGMA = 2.0
GROUPS = 32
NEG_SLOPE = 0.1
EPS = 1e-5

NW = 32
PPW = 320
NPAD = NW * PPW
KGROUPS = ((0, 4), (4, 4), (8, 4), (12, 3))


def _sqrt16(x):
    xi = lax.bitcast_convert_type(x, jnp.int32)
    yi = jnp.int32(0x5F3759DF) - (xi >> 1)
    y = lax.bitcast_convert_type(yi, jnp.float32)
    for _ in range(2):
        y = y * (1.5 - 0.5 * x * y * y)
    return x * y


def _sc_body(feats, px, py, pz, rs, qp, idxp, kp,
             out_w, out_nn,
             px_v, py_v, pz_v, r_v, q_v, idx_v, kp_v, w_v, nn_v,
             rows0, rows1, wacc0, wacc1, gsem0, gsem1, osem0, osem1):
    wid = lax.axis_index("s") * 2 + lax.axis_index("c")
    base = wid * PPW
    pltpu.sync_copy(px, px_v)
    pltpu.sync_copy(py, py_v)
    pltpu.sync_copy(pz, pz_v)
    pltpu.sync_copy(rs, r_v)
    pltpu.sync_copy(qp.at[pl.ds(base * 16, PPW * 16)], q_v)
    pltpu.sync_copy(idxp.at[pl.ds(base * H, PPW * H)], idx_v.at[pl.ds(0, PPW * H)])
    pltpu.sync_copy(kp, kp_v)
    zi = jnp.zeros((16,), jnp.int32)
    idx_v[pl.ds(PPW * H, 16)] = zi
    idx_v[pl.ds(PPW * H + 16, 16)] = zi

    rows = (rows0, rows1)
    waccs = (wacc0, wacc1)
    gsems = (gsem0, gsem1)
    osems = (osem0, osem1)

    pltpu.async_copy(feats.at[idx_v.at[pl.ds(0, H)]], rows0, gsem0)

    def do_point(i, b):
        rows_b = rows[b]
        wacc_b = waccs[b]
        pltpu.async_copy(feats.at[idx_v.at[pl.ds((i + 1) * H, H)]],
                         rows[1 - b], gsems[1 - b])

        iv0 = idx_v[pl.ds(i * H, 16)]
        iv1 = idx_v[pl.ds(i * H + 16, 16)]
        qrow = q_v[pl.ds(i * 16, 16)]
        qx = qrow[0]
        qy = qrow[1]
        qz = qrow[2]
        nx0 = plsc.load_gather(px_v, [iv0]) - qx
        ny0 = plsc.load_gather(py_v, [iv0]) - qy
        nz0 = plsc.load_gather(pz_v, [iv0]) - qz
        nx1 = plsc.load_gather(px_v, [iv1]) - qx
        ny1 = plsc.load_gather(py_v, [iv1]) - qy
        nz1 = plsc.load_gather(pz_v, [iv1]) - qz

        rs0 = plsc.load_gather(r_v, [iv0])
        rs1 = plsc.load_gather(r_v, [iv1])
        cnt = (plsc.all_reduce_population_count(rs0 > 0.0)
               + plsc.all_reduce_population_count(rs1 > 0.0))
        cntf = jnp.maximum(cnt.astype(jnp.float32), 1.0)
        nn_v[pl.ds(i * 16, 16)] = cntf

        kpxv = kp_v[pl.ds(0, 16)]
        kpyv = kp_v[pl.ds(16, 16)]
        kpzv = kp_v[pl.ds(32, 16)]
        lane = lax.broadcasted_iota(jnp.int32, (16,), 0)
        for k in range(K):
            kpx = kpxv[k]
            kpy = kpyv[k]
            kpz = kpzv[k]
            kcol = jnp.full((16,), k, jnp.int32)
            for g, (ax, ay, az) in enumerate(((nx0, ny0, nz0), (nx1, ny1, nz1))):
                dx = ax - kpx
                dy = ay - kpy
                dz = az - kpz
                sq = dx * dx + dy * dy + dz * dz
                dd = _sqrt16(sq)
                w = jnp.maximum(1.0 - dd * (1.0 / SIGMA), 0.0)
                plsc.store_scatter(w_v, [(lane + g * 16) * 16 + kcol], w)

        pltpu.make_async_copy(feats.at[idx_v.at[pl.ds(i * H, H)]],
                              rows_b, gsems[b]).wait()

        for k0, ksz in KGROUPS:
            def hbody(h, acc, _k0=k0, _ksz=ksz):
                wrow = w_v[pl.ds(h * 16, 16)]
                ws = tuple(jnp.full((16,), wrow[_k0 + t2], jnp.float32)
                           for t2 in range(_ksz))
                rvs = tuple(rows_b[h, pl.ds(j * 16, 16)] for j in range(8))
                return tuple(acc[t2 * 8 + j] + ws[t2] * rvs[j]
                             for t2 in range(_ksz) for j in range(8))

            acc0 = tuple(jnp.zeros((16,), jnp.float32) for _ in range(ksz * 8))
            acc = lax.fori_loop(0, H, hbody, acc0)
            for t in range(ksz):
                for j in range(8):
                    wacc_b[pl.ds((k0 + t) * CIN + j * 16, 16)] = \
                        acc[t * 8 + j]

        @pl.when(i >= 2)
        def _():
            pltpu.make_async_copy(
                wacc_b, out_w.at[pl.ds((base + i - 2) * K * CIN, K * CIN)],
                osems[b]).wait()

        pltpu.async_copy(
            wacc_b, out_w.at[pl.ds((base + i) * K * CIN, K * CIN)], osems[b])

    def loop_body(i2, carry):
        do_point(i2 * 2, 0)
        do_point(i2 * 2 + 1, 1)
        return carry

    lax.fori_loop(0, PPW // 2, loop_body, 0)

    pltpu.make_async_copy(
        wacc0, out_w.at[pl.ds((base + PPW - 2) * K * CIN, K * CIN)],
        osem0).wait()
    pltpu.make_async_copy(
        wacc1, out_w.at[pl.ds((base + PPW - 1) * K * CIN, K * CIN)],
        osem1).wait()
    pltpu.make_async_copy(feats.at[idx_v.at[pl.ds(PPW * H, H)]],
                          rows0, gsem0).wait()
    pltpu.sync_copy(nn_v, out_nn.at[pl.ds(base * 16, PPW * 16)])


def _sc_stage(feats, px, py, pz, rsum, qpad, idxpad, kp_soa):
    mesh = plsc.VectorSubcoreMesh(core_axis_name="c", subcore_axis_name="s")
    f = pl.kernel(
        _sc_body,
        mesh=mesh,
        compiler_params=pltpu.CompilerParams(needs_layout_passes=False),
        out_type=[
            jax.ShapeDtypeStruct((NPAD * K * CIN,), jnp.float32),
            jax.ShapeDtypeStruct((NPAD * 16,), jnp.float32),
        ],
        scratch_types=[
            pltpu.VMEM((N,), jnp.float32),
            pltpu.VMEM((N,), jnp.float32),
            pltpu.VMEM((N,), jnp.float32),
            pltpu.VMEM((N,), jnp.float32),
            pltpu.VMEM((PPW * 16,), jnp.float32),
            pltpu.VMEM(((PPW + 1) * H,), jnp.int32),
            pltpu.VMEM((48,), jnp.float32),
            pltpu.VMEM((H * 16,), jnp.float32),
            pltpu.VMEM((PPW * 16,), jnp.float32),
            pltpu.VMEM((H, CIN), jnp.float32),
            pltpu.VMEM((H, CIN), jnp.float32),
            pltpu.VMEM((K * CIN,), jnp.float32),
            pltpu.VMEM((K * CIN,), jnp.float32),
            pltpu.SemaphoreType.DMA,
            pltpu.SemaphoreType.DMA,
            pltpu.SemaphoreType.DMA,
            pltpu.SemaphoreType.DMA,
        ],
    )
    return f(feats, px, py, pz, rsum, qpad, idxpad, kp_soa)


def _rowsum_stage(feats):
    def body(f_ref, r_ref):
        r_ref[...] = jnp.sum(f_ref[...], axis=1, keepdims=True)

    return pl.pallas_call(
        body,
        out_shape=jax.ShapeDtypeStruct((N, 1), jnp.float32),
    )(feats)


def _mm_stats_stage(w2d, wf, nn, bias2d):
    B = 2048
    nblk = NPAD // B

    def body(w_ref, wf_ref, nn_ref, b_ref, x_ref, s_ref, q_ref):
        i = pl.program_id(0)
        mm = jnp.dot(w_ref[...], wf_ref[...],
                     preferred_element_type=jnp.float32)
        x = mm / nn_ref[...] + b_ref[...]
        x_ref[...] = x
        gid = i * B + lax.broadcasted_iota(jnp.int32, (B, 1), 0)
        xm = jnp.where(gid < N, x, 0.0)

        @pl.when(i == 0)
        def _():
            s_ref[...] = jnp.zeros_like(s_ref)
            q_ref[...] = jnp.zeros_like(q_ref)

        s_ref[...] += jnp.sum(xm, axis=0, keepdims=True)
        q_ref[...] += jnp.sum(xm * xm, axis=0, keepdims=True)

    return pl.pallas_call(
        body,
        grid=(nblk,),
        in_specs=[
            pl.BlockSpec((B, K * CIN), lambda i: (i, 0)),
            pl.BlockSpec((K * CIN, COUT), lambda i: (0, 0)),
            pl.BlockSpec((B, 1), lambda i: (i, 0)),
            pl.BlockSpec((1, COUT), lambda i: (0, 0)),
        ],
        out_specs=[
            pl.BlockSpec((B, COUT), lambda i: (i, 0)),
            pl.BlockSpec((1, COUT), lambda i: (0, 0)),
            pl.BlockSpec((1, COUT), lambda i: (0, 0)),
        ],
        out_shape=[
            jax.ShapeDtypeStruct((NPAD, COUT), jnp.float32),
            jax.ShapeDtypeStruct((1, COUT), jnp.float32),
            jax.ShapeDtypeStruct((1, COUT), jnp.float32),
        ],
    )(w2d, wf, nn, bias2d)


def _norm_stage(x, s, q, gamma2d, beta2d):
    B = 2000

    def body(x_ref, s_ref, q_ref, g_ref, b_ref, y_ref):
        gsz = COUT // GROUPS
        ii = lax.broadcasted_iota(jnp.int32, (COUT, COUT), 0) // gsz
        jj = lax.broadcasted_iota(jnp.int32, (COUT, COUT), 1) // gsz
        gmat = (ii == jj).astype(jnp.float32)
        denom = float(gsz * N)
        mean = jnp.dot(s_ref[...], gmat,
                       preferred_element_type=jnp.float32) / denom
        e2 = jnp.dot(q_ref[...], gmat,
                     preferred_element_type=jnp.float32) / denom
        var = e2 - mean * mean
        scale = g_ref[...] * lax.rsqrt(var + EPS)
        shift = b_ref[...] - mean * scale
        y = x_ref[...] * scale + shift
        y_ref[...] = jnp.where(y >= 0.0, y, NEG_SLOPE * y)

    return pl.pallas_call(
        body,
        grid=(N // B,),
        in_specs=[
            pl.BlockSpec((B, COUT), lambda i: (i, 0)),
            pl.BlockSpec((1, COUT), lambda i: (0, 0)),
            pl.BlockSpec((1, COUT), lambda i: (0, 0)),
            pl.BlockSpec((1, COUT), lambda i: (0, 0)),
            pl.BlockSpec((1, COUT), lambda i: (0, 0)),
        ],
        out_specs=pl.BlockSpec((B, COUT), lambda i: (i, 0)),
        out_shape=jax.ShapeDtypeStruct((N, COUT), jnp.float32),
    )(x, s, q, gamma2d, beta2d)


def kernel(s_feats, q_points, s_points, neighbor_indices, weights, bias,
           gamma, beta, kernel_points):
    f32 = jnp.float32
    rsum = _rowsum_stage(s_feats).reshape(N)

    px = s_points[:, 0].astype(f32)
    py = s_points[:, 1].astype(f32)
    pz = s_points[:, 2].astype(f32)
    qpad = jnp.pad(q_points.astype(f32), ((0, NPAD - N), (0, 13))).reshape(-1)
    idxpad = jnp.concatenate(
        [neighbor_indices.astype(jnp.int32),
         jnp.zeros((NPAD - N, H), jnp.int32)], axis=0).reshape(-1)
    kp_soa = jnp.concatenate(
        [kernel_points.astype(f32).T, jnp.zeros((3, 16 - K), f32)],
        axis=1).reshape(-1)

    weighted, nnflat = _sc_stage(s_feats.astype(f32), px, py, pz, rsum,
                                 qpad, idxpad, kp_soa)

    w2d = weighted.reshape(NPAD, K * CIN)
    wf = weights.astype(f32).reshape(K * CIN, COUT)
    nn = nnflat.reshape(NPAD, 16)[:, :1]
    x, s, q = _mm_stats_stage(w2d, wf, nn, bias.astype(f32).reshape(1, COUT))
    y = _norm_stage(x, s, q, gamma.astype(f32).reshape(1, COUT),
                    beta.astype(f32).reshape(1, COUT))
    return y[:, None, :]
